# initial kernel scaffold (unmeasured)
import jax
import jax.numpy as jnp
from jax import lax
from jax.experimental import pallas as pl
from jax.experimental.pallas import tpu as pltpu

N_DEV = 4
N_EXP = 16


def kernel(x, router_W, route_idx, expert_W):
    n_tok, d = x.shape
    e_per, _, h = expert_W.shape

    def body(x_ref, rw_ref, idx_ref, ew_ref, out_ref, comm_ref, send_sems, recv_sems):
        my = lax.axis_index("i")
        left = lax.rem(my + (N_DEV - 1), N_DEV)
        right = lax.rem(my + 1, N_DEV)

        comm_ref[0, :, :, :] = ew_ref[:, :, :]

        barrier_sem = pltpu.get_barrier_semaphore()
        for nbr in (left, right):
            pl.semaphore_signal(
                barrier_sem, inc=1,
                device_id=(nbr,), device_id_type=pl.DeviceIdType.MESH,
            )
        pl.semaphore_wait(barrier_sem, 2)

        xf = x_ref[:, :]
        scores = jnp.dot(xf, rw_ref[:, :], preferred_element_type=jnp.float32)
        smax = jnp.max(scores, axis=-1, keepdims=True)
        p = jnp.exp(scores - smax)
        p = p / jnp.sum(p, axis=-1, keepdims=True)
        e_ids = lax.broadcasted_iota(jnp.int32, (n_tok, N_EXP), 1)
        sel = (idx_ref[:, 0:1] == e_ids) | (idx_ref[:, 1:2] == e_ids)
        gated = jnp.where(sel, p, 0.0)
        wgt = gated / jnp.sum(gated, axis=-1, keepdims=True)

        x_bf = xf.astype(jnp.bfloat16)

        def contrib(origin, slot):
            acc = jnp.zeros((n_tok, h), jnp.float32)
            for j in range(e_per):
                e = origin * e_per + j
                g = jnp.sum(jnp.where(e_ids == e, wgt, 0.0), axis=1, keepdims=True)
                xg = x_bf * g.astype(jnp.bfloat16)
                acc = acc + jnp.dot(
                    xg, comm_ref[slot, j, :, :], preferred_element_type=jnp.float32
                )
            return acc

        acc = jnp.zeros((n_tok, h), jnp.float32)
        for hop in range(N_DEV - 1):
            s_slot = hop % 2
            r_slot = (hop + 1) % 2
            rdma = pltpu.make_async_remote_copy(
                src_ref=comm_ref.at[s_slot],
                dst_ref=comm_ref.at[r_slot],
                send_sem=send_sems.at[s_slot],
                recv_sem=recv_sems.at[r_slot],
                device_id=(right,),
                device_id_type=pl.DeviceIdType.MESH,
            )
            rdma.start()
            origin = lax.rem(my + (N_DEV - hop), N_DEV)
            acc = acc + contrib(origin, s_slot)
            rdma.wait()
        acc = acc + contrib(lax.rem(my + 1, N_DEV), (N_DEV - 1) % 2)
        out_ref[:, :] = acc

    return pl.pallas_call(
        body,
        out_shape=jax.ShapeDtypeStruct((n_tok, h), jnp.float32),
        in_specs=[
            pl.BlockSpec(memory_space=pltpu.VMEM),
            pl.BlockSpec(memory_space=pltpu.VMEM),
            pl.BlockSpec(memory_space=pltpu.VMEM),
            pl.BlockSpec(memory_space=pltpu.VMEM),
        ],
        out_specs=pl.BlockSpec(memory_space=pltpu.VMEM),
        scratch_shapes=[
            pltpu.VMEM((2, e_per, d, h), jnp.bfloat16),
            pltpu.SemaphoreType.DMA((2,)),
            pltpu.SemaphoreType.DMA((2,)),
        ],
        compiler_params=pltpu.CompilerParams(collective_id=0),
    )(x, router_W, route_idx, expert_W.astype(jnp.bfloat16))


# baseline (device time: 165331 ns/iter reference)
import jax
import jax.numpy as jnp
from jax import lax
from jax.experimental import pallas as pl
from jax.experimental.pallas import tpu as pltpu

N_DEV = 4
N_EXP = 16


def kernel(x, router_W, route_idx, expert_W):
    n_tok, d = x.shape
    e_per, _, h = expert_W.shape

    def body(x_ref, rw_ref, idx_ref, ew_ref, out_ref,
             comm_ref, wgt_ref, xbf_ref, send_sems, recv_sems):
        my = lax.axis_index("i")
        left = lax.rem(my + (N_DEV - 1), N_DEV)
        right = lax.rem(my + 1, N_DEV)

        comm_ref[0, :, :, :] = ew_ref[:, :, :]

        barrier_sem = pltpu.get_barrier_semaphore()
        for nbr in (left, right):
            pl.semaphore_signal(
                barrier_sem, inc=1,
                device_id=(nbr,), device_id_type=pl.DeviceIdType.MESH,
            )
        pl.semaphore_wait(barrier_sem, 2)

        scores = jnp.dot(x_ref[:, :], rw_ref[:, :],
                         preferred_element_type=jnp.float32)
        smax = jnp.max(scores, axis=-1, keepdims=True)
        p = jnp.exp(scores - smax)
        p = p / jnp.sum(p, axis=-1, keepdims=True)
        e_ids = lax.broadcasted_iota(jnp.int32, (n_tok, N_EXP), 1)
        sel = (idx_ref[:, 0:1] == e_ids) | (idx_ref[:, 1:2] == e_ids)
        gated = jnp.where(sel, p, 0.0)
        wgt_ref[:, :] = gated / jnp.sum(gated, axis=-1, keepdims=True)
        xbf_ref[:, :] = x_ref[:, :].astype(jnp.bfloat16)
        out_ref[:, :] = jnp.zeros((n_tok, h), jnp.float32)

        def contrib(origin, slot):
            ids = lax.broadcasted_iota(jnp.int32, (n_tok, N_EXP), 1)
            for j in range(e_per):
                e = origin * e_per + j
                g = jnp.sum(jnp.where(ids == e, wgt_ref[:, :], 0.0),
                            axis=1, keepdims=True)
                xg = xbf_ref[:, :] * g.astype(jnp.bfloat16)
                out_ref[:, :] = out_ref[:, :] + jnp.dot(
                    xg, comm_ref[slot, j, :, :],
                    preferred_element_type=jnp.float32)

        for hop in range(N_DEV - 1):
            s_slot = hop % 2
            r_slot = (hop + 1) % 2
            rdma = pltpu.make_async_remote_copy(
                src_ref=comm_ref.at[s_slot],
                dst_ref=comm_ref.at[r_slot],
                send_sem=send_sems.at[s_slot],
                recv_sem=recv_sems.at[r_slot],
                device_id=(right,),
                device_id_type=pl.DeviceIdType.MESH,
            )
            rdma.start()
            contrib(lax.rem(my + (N_DEV - hop), N_DEV), s_slot)
            rdma.wait()
        contrib(lax.rem(my + 1, N_DEV), (N_DEV - 1) % 2)

    return pl.pallas_call(
        body,
        out_shape=jax.ShapeDtypeStruct((n_tok, h), jnp.float32),
        in_specs=[
            pl.BlockSpec(memory_space=pltpu.VMEM),
            pl.BlockSpec(memory_space=pltpu.VMEM),
            pl.BlockSpec(memory_space=pltpu.VMEM),
            pl.BlockSpec(memory_space=pltpu.VMEM),
        ],
        out_specs=pl.BlockSpec(memory_space=pltpu.VMEM),
        scratch_shapes=[
            pltpu.VMEM((2, e_per, d, h), jnp.bfloat16),
            pltpu.VMEM((n_tok, N_EXP), jnp.float32),
            pltpu.VMEM((n_tok, d), jnp.bfloat16),
            pltpu.SemaphoreType.DMA((2,)),
            pltpu.SemaphoreType.DMA((2,)),
        ],
        compiler_params=pltpu.CompilerParams(collective_id=0),
    )(x, router_W, route_idx, expert_W.astype(jnp.bfloat16))


# device time: 92455 ns/iter; 1.7882x vs baseline; 1.7882x over previous
import jax
import jax.numpy as jnp
from jax import lax
from jax.experimental import pallas as pl
from jax.experimental.pallas import tpu as pltpu

N_DEV = 4
N_EXP = 16


def kernel(x, router_W, route_idx, expert_W):
    n_tok, d = x.shape
    e_per, _, h = expert_W.shape
    half = e_per // 2

    def body(x_ref, rw_ref, idx_ref, ew_ref, out_ref,
             comm_ref, wgt_ref, xbf_ref, send_sems, recv_sems):
        my = lax.axis_index("i")
        left = lax.rem(my + (N_DEV - 1), N_DEV)
        right = lax.rem(my + 1, N_DEV)

        comm_ref[0, :, :, :] = ew_ref[:, :, :]

        barrier_sem = pltpu.get_barrier_semaphore()
        for nbr in (left, right):
            pl.semaphore_signal(
                barrier_sem, inc=1,
                device_id=(nbr,), device_id_type=pl.DeviceIdType.MESH,
            )
        pl.semaphore_wait(barrier_sem, 2)

        def copy(src, dst, s_sem, r_sem, dev):
            return pltpu.make_async_remote_copy(
                src_ref=src, dst_ref=dst,
                send_sem=send_sems.at[s_sem], recv_sem=recv_sems.at[r_sem],
                device_id=(dev,), device_id_type=pl.DeviceIdType.MESH,
            )

        a_right = copy(comm_ref.at[0], comm_ref.at[1], 0, 1, right)
        a_left = copy(comm_ref.at[0], comm_ref.at[2], 1, 2, left)
        a_right.start()
        a_left.start()

        recv_l = copy(comm_ref.at[1], comm_ref.at[1], 0, 1, left)
        recv_r = copy(comm_ref.at[2], comm_ref.at[2], 0, 2, right)
        opp_a = copy(comm_ref.at[3, pl.ds(0, half)],
                     comm_ref.at[3, pl.ds(0, half)], 0, 3, left)
        opp_b = copy(comm_ref.at[3, pl.ds(half, half)],
                     comm_ref.at[3, pl.ds(half, half)], 0, 0, right)

        scores = jnp.dot(x_ref[:, :], rw_ref[:, :],
                         preferred_element_type=jnp.float32)
        smax = jnp.max(scores, axis=-1, keepdims=True)
        p = jnp.exp(scores - smax)
        p = p / jnp.sum(p, axis=-1, keepdims=True)
        e_ids = lax.broadcasted_iota(jnp.int32, (n_tok, N_EXP), 1)
        sel = (idx_ref[:, 0:1] == e_ids) | (idx_ref[:, 1:2] == e_ids)
        gated = jnp.where(sel, p, 0.0)
        wgt_ref[:, :] = gated / jnp.sum(gated, axis=-1, keepdims=True)
        xbf_ref[:, :] = x_ref[:, :].astype(jnp.bfloat16)
        out_ref[:, :] = jnp.zeros((n_tok, h), jnp.float32)

        def contrib(origin, slot):
            ids = lax.broadcasted_iota(jnp.int32, (n_tok, N_EXP), 1)
            for j in range(e_per):
                e = origin * e_per + j
                g = jnp.sum(jnp.where(ids == e, wgt_ref[:, :], 0.0),
                            axis=1, keepdims=True)
                xg = xbf_ref[:, :] * g.astype(jnp.bfloat16)
                out_ref[:, :] = out_ref[:, :] + jnp.dot(
                    xg, comm_ref[slot, j, :, :],
                    preferred_element_type=jnp.float32)

        contrib(my, 0)

        recv_l.wait_recv()
        b_right = copy(comm_ref.at[1, pl.ds(0, half)],
                       comm_ref.at[3, pl.ds(0, half)], 2, 3, right)
        b_right.start()
        recv_r.wait_recv()
        b_left = copy(comm_ref.at[2, pl.ds(half, half)],
                      comm_ref.at[3, pl.ds(half, half)], 3, 0, left)
        b_left.start()

        contrib(left, 1)
        contrib(right, 2)

        opp_a.wait_recv()
        opp_b.wait_recv()
        contrib(lax.rem(my + 2, N_DEV), 3)

        a_right.wait_send()
        a_left.wait_send()
        b_right.wait_send()
        b_left.wait_send()

    return pl.pallas_call(
        body,
        out_shape=jax.ShapeDtypeStruct((n_tok, h), jnp.float32),
        in_specs=[
            pl.BlockSpec(memory_space=pltpu.VMEM),
            pl.BlockSpec(memory_space=pltpu.VMEM),
            pl.BlockSpec(memory_space=pltpu.VMEM),
            pl.BlockSpec(memory_space=pltpu.VMEM),
        ],
        out_specs=pl.BlockSpec(memory_space=pltpu.VMEM),
        scratch_shapes=[
            pltpu.VMEM((N_DEV, e_per, d, h), jnp.bfloat16),
            pltpu.VMEM((n_tok, N_EXP), jnp.float32),
            pltpu.VMEM((n_tok, d), jnp.bfloat16),
            pltpu.SemaphoreType.DMA((4,)),
            pltpu.SemaphoreType.DMA((4,)),
        ],
        compiler_params=pltpu.CompilerParams(collective_id=0),
    )(x, router_W, route_idx, expert_W.astype(jnp.bfloat16))


# device time: 92448 ns/iter; 1.7884x vs baseline; 1.0001x over previous
import jax
import jax.numpy as jnp
from jax import lax
from jax.experimental import pallas as pl
from jax.experimental.pallas import tpu as pltpu

N_DEV = 4
N_EXP = 16


def kernel(x, router_W, route_idx, expert_W):
    n_tok, d = x.shape
    e_per, _, h = expert_W.shape
    half = e_per // 2

    def body(x_ref, rw_ref, idx_ref, ew_ref, out_ref,
             comm_ref, wgt_ref, xbf_ref, send_sems, recv_sems):
        my = lax.axis_index("i")
        left = lax.rem(my + (N_DEV - 1), N_DEV)
        right = lax.rem(my + 1, N_DEV)

        comm_ref[0, :, :, :] = ew_ref[:, :, :]

        barrier_sem = pltpu.get_barrier_semaphore()
        for nbr in (left, right):
            pl.semaphore_signal(
                barrier_sem, inc=1,
                device_id=(nbr,), device_id_type=pl.DeviceIdType.MESH,
            )
        pl.semaphore_wait(barrier_sem, 2)

        def copy(src, dst, s_sem, r_sem, dev):
            return pltpu.make_async_remote_copy(
                src_ref=src, dst_ref=dst,
                send_sem=send_sems.at[s_sem], recv_sem=recv_sems.at[r_sem],
                device_id=(dev,), device_id_type=pl.DeviceIdType.MESH,
            )

        a_right = copy(comm_ref.at[0], comm_ref.at[1], 0, 1, right)
        a_left = copy(comm_ref.at[0], comm_ref.at[2], 1, 2, left)
        a_right.start()
        a_left.start()

        recv_l = copy(comm_ref.at[1], comm_ref.at[1], 0, 1, left)
        recv_r = copy(comm_ref.at[2], comm_ref.at[2], 0, 2, right)
        opp_a = copy(comm_ref.at[3, pl.ds(0, half)],
                     comm_ref.at[3, pl.ds(0, half)], 0, 3, left)
        opp_b = copy(comm_ref.at[3, pl.ds(half, half)],
                     comm_ref.at[3, pl.ds(half, half)], 0, 0, right)

        scores = jnp.dot(x_ref[:, :], rw_ref[:, :],
                         preferred_element_type=jnp.float32)
        smax = jnp.max(scores, axis=-1, keepdims=True)
        p = jnp.exp(scores - smax)
        p = p / jnp.sum(p, axis=-1, keepdims=True)
        e_ids = lax.broadcasted_iota(jnp.int32, (n_tok, N_EXP), 1)
        sel = (idx_ref[:, 0:1] == e_ids) | (idx_ref[:, 1:2] == e_ids)
        gated = jnp.where(sel, p, 0.0)
        wgt_ref[:, :] = gated / jnp.sum(gated, axis=-1, keepdims=True)
        xbf_ref[:, :] = x_ref[:, :].astype(jnp.bfloat16)
        out_ref[:, :] = jnp.zeros((n_tok, h), jnp.float32)

        def contrib(origin, slot):
            ids = lax.broadcasted_iota(jnp.int32, (n_tok, N_EXP), 1)
            parts = []
            for j in range(e_per):
                e = origin * e_per + j
                g = jnp.sum(jnp.where(ids == e, wgt_ref[:, :], 0.0),
                            axis=1, keepdims=True)
                parts.append(xbf_ref[:, :] * g.astype(jnp.bfloat16))
            xg = jnp.concatenate(parts, axis=1)
            w = comm_ref[slot, :, :, :].reshape(e_per * d, h)
            out_ref[:, :] = out_ref[:, :] + jnp.dot(
                xg, w, preferred_element_type=jnp.float32)

        contrib(my, 0)

        recv_l.wait_recv()
        b_right = copy(comm_ref.at[1, pl.ds(0, half)],
                       comm_ref.at[3, pl.ds(0, half)], 2, 3, right)
        b_right.start()
        recv_r.wait_recv()
        b_left = copy(comm_ref.at[2, pl.ds(half, half)],
                      comm_ref.at[3, pl.ds(half, half)], 3, 0, left)
        b_left.start()

        contrib(left, 1)
        contrib(right, 2)

        opp_a.wait_recv()
        opp_b.wait_recv()
        contrib(lax.rem(my + 2, N_DEV), 3)

        a_right.wait_send()
        a_left.wait_send()
        b_right.wait_send()
        b_left.wait_send()

    return pl.pallas_call(
        body,
        out_shape=jax.ShapeDtypeStruct((n_tok, h), jnp.float32),
        in_specs=[
            pl.BlockSpec(memory_space=pltpu.VMEM),
            pl.BlockSpec(memory_space=pltpu.VMEM),
            pl.BlockSpec(memory_space=pltpu.VMEM),
            pl.BlockSpec(memory_space=pltpu.VMEM),
        ],
        out_specs=pl.BlockSpec(memory_space=pltpu.VMEM),
        scratch_shapes=[
            pltpu.VMEM((N_DEV, e_per, d, h), jnp.bfloat16),
            pltpu.VMEM((n_tok, N_EXP), jnp.float32),
            pltpu.VMEM((n_tok, d), jnp.bfloat16),
            pltpu.SemaphoreType.DMA((4,)),
            pltpu.SemaphoreType.DMA((4,)),
        ],
        compiler_params=pltpu.CompilerParams(collective_id=0),
    )(x, router_W, route_idx, expert_W.astype(jnp.bfloat16))


# device time: 87120 ns/iter; 1.8977x vs baseline; 1.0612x over previous
import jax
import jax.numpy as jnp
from jax import lax
from jax.experimental import pallas as pl
from jax.experimental.pallas import tpu as pltpu

N_DEV = 4
N_EXP = 16


def kernel(x, router_W, route_idx, expert_W):
    n_tok, d = x.shape
    e_per, _, h = expert_W.shape

    def body(x_ref, rw_ref, idx_ref, ew_ref, out_ref,
             comm_ref, wgt_ref, xbf_ref, send_sems, recv_sems):
        my = lax.axis_index("i")
        left = lax.rem(my + (N_DEV - 1), N_DEV)
        right = lax.rem(my + 1, N_DEV)

        comm_ref[0, :, :, :] = ew_ref[:, :, :]

        barrier_sem = pltpu.get_barrier_semaphore()
        for nbr in (left, right):
            pl.semaphore_signal(
                barrier_sem, inc=1,
                device_id=(nbr,), device_id_type=pl.DeviceIdType.MESH,
            )
        pl.semaphore_wait(barrier_sem, 2)

        def copy(src_slot, src_j, dst_slot, dst_j, sem, dev):
            return pltpu.make_async_remote_copy(
                src_ref=comm_ref.at[src_slot, src_j],
                dst_ref=comm_ref.at[dst_slot, dst_j],
                send_sem=send_sems.at[sem], recv_sem=recv_sems.at[sem],
                device_id=(dev,), device_id_type=pl.DeviceIdType.MESH,
            )

        a_sends = []
        for j in range(e_per):
            ar = copy(0, j, 1, j, j, right)
            ar.start()
            a_sends.append(ar)
        for j in reversed(range(e_per)):
            al = copy(0, j, 2, j, 4 + j, left)
            al.start()
            a_sends.append(al)

        recv_l = [copy(1, j, 1, j, j, left) for j in range(e_per)]
        recv_r = [copy(2, j, 2, j, 4 + j, right) for j in range(e_per)]
        recv_opp = [copy(3, j, 3, j, 8 + j, left) for j in range(e_per)]

        scores = jnp.dot(x_ref[:, :], rw_ref[:, :],
                         preferred_element_type=jnp.float32)
        smax = jnp.max(scores, axis=-1, keepdims=True)
        p = jnp.exp(scores - smax)
        p = p / jnp.sum(p, axis=-1, keepdims=True)
        e_ids = lax.broadcasted_iota(jnp.int32, (n_tok, N_EXP), 1)
        sel = (idx_ref[:, 0:1] == e_ids) | (idx_ref[:, 1:2] == e_ids)
        gated = jnp.where(sel, p, 0.0)
        wgt_ref[:, :] = gated / jnp.sum(gated, axis=-1, keepdims=True)
        xbf_ref[:, :] = x_ref[:, :].astype(jnp.bfloat16)
        out_ref[:, :] = jnp.zeros((n_tok, h), jnp.float32)

        def contrib1(origin, slot, j):
            ids = lax.broadcasted_iota(jnp.int32, (n_tok, N_EXP), 1)
            e = origin * e_per + j
            g = jnp.sum(jnp.where(ids == e, wgt_ref[:, :], 0.0),
                        axis=1, keepdims=True)
            xg = xbf_ref[:, :] * g.astype(jnp.bfloat16)
            out_ref[:, :] = out_ref[:, :] + jnp.dot(
                xg, comm_ref[slot, j, :, :], preferred_element_type=jnp.float32)

        for j in range(e_per):
            contrib1(my, 0, j)

        b_sends = []
        for j in (0, 1):
            recv_l[j].wait_recv()
            br = copy(1, j, 3, j, 8 + j, right)
            br.start()
            b_sends.append(br)
        for j in (3, 2):
            recv_r[j].wait_recv()
            bl = copy(2, j, 3, j, 8 + j, left)
            bl.start()
            b_sends.append(bl)

        opp = lax.rem(my + 2, N_DEV)
        contrib1(left, 1, 0)
        contrib1(right, 2, 3)
        contrib1(left, 1, 1)
        contrib1(right, 2, 2)
        recv_l[2].wait_recv()
        contrib1(left, 1, 2)
        recv_r[1].wait_recv()
        contrib1(right, 2, 1)
        recv_l[3].wait_recv()
        contrib1(left, 1, 3)
        recv_r[0].wait_recv()
        contrib1(right, 2, 0)
        recv_opp[0].wait_recv()
        contrib1(opp, 3, 0)
        recv_opp[3].wait_recv()
        contrib1(opp, 3, 3)
        recv_opp[1].wait_recv()
        contrib1(opp, 3, 1)
        recv_opp[2].wait_recv()
        contrib1(opp, 3, 2)

        for s in a_sends + b_sends:
            s.wait_send()

    return pl.pallas_call(
        body,
        out_shape=jax.ShapeDtypeStruct((n_tok, h), jnp.float32),
        in_specs=[
            pl.BlockSpec(memory_space=pltpu.VMEM),
            pl.BlockSpec(memory_space=pltpu.VMEM),
            pl.BlockSpec(memory_space=pltpu.VMEM),
            pl.BlockSpec(memory_space=pltpu.VMEM),
        ],
        out_specs=pl.BlockSpec(memory_space=pltpu.VMEM),
        scratch_shapes=[
            pltpu.VMEM((N_DEV, e_per, d, h), jnp.bfloat16),
            pltpu.VMEM((n_tok, N_EXP), jnp.float32),
            pltpu.VMEM((n_tok, d), jnp.bfloat16),
            pltpu.SemaphoreType.DMA((12,)),
            pltpu.SemaphoreType.DMA((12,)),
        ],
        compiler_params=pltpu.CompilerParams(collective_id=0),
    )(x, router_W, route_idx, expert_W.astype(jnp.bfloat16))


# device time: 87065 ns/iter; 1.8989x vs baseline; 1.0006x over previous
import jax
import jax.numpy as jnp
from jax import lax
from jax.experimental import pallas as pl
from jax.experimental.pallas import tpu as pltpu

N_DEV = 4
N_EXP = 16


def kernel(x, router_W, route_idx, expert_W):
    n_tok, d = x.shape
    e_per, _, h = expert_W.shape

    def body(x_ref, rw_ref, idx_ref, ew_ref, out_ref,
             comm_ref, wgt_ref, xbf_ref, send_sems, recv_sems):
        my = lax.axis_index("i")
        left = lax.rem(my + (N_DEV - 1), N_DEV)
        right = lax.rem(my + 1, N_DEV)

        comm_ref[0, :, :, :] = ew_ref[:, :, :]

        barrier_sem = pltpu.get_barrier_semaphore()
        for nbr in (left, right):
            pl.semaphore_signal(
                barrier_sem, inc=1,
                device_id=(nbr,), device_id_type=pl.DeviceIdType.MESH,
            )
        pl.semaphore_wait(barrier_sem, 2)

        def copy(src_slot, src_j, dst_slot, dst_j, sem, dev):
            return pltpu.make_async_remote_copy(
                src_ref=comm_ref.at[src_slot, src_j],
                dst_ref=comm_ref.at[dst_slot, dst_j],
                send_sem=send_sems.at[sem], recv_sem=recv_sems.at[sem],
                device_id=(dev,), device_id_type=pl.DeviceIdType.MESH,
            )

        a_sends = []
        for j in range(e_per):
            ar = copy(0, j, 1, j, j, right)
            ar.start()
            a_sends.append(ar)
        for j in reversed(range(e_per)):
            al = copy(0, j, 2, j, 4 + j, left)
            al.start()
            a_sends.append(al)

        recv_l = [copy(1, j, 1, j, j, left) for j in range(e_per)]
        recv_r = [copy(2, j, 2, j, 4 + j, right) for j in range(e_per)]
        recv_opp = [copy(3, j, 3, j, 8 + j, left) for j in range(e_per)]

        scores = jnp.dot(x_ref[:, :], rw_ref[:, :],
                         preferred_element_type=jnp.float32)
        smax = jnp.max(scores, axis=-1, keepdims=True)
        p = jnp.exp(scores - smax)
        p = p / jnp.sum(p, axis=-1, keepdims=True)
        e_ids = lax.broadcasted_iota(jnp.int32, (n_tok, N_EXP), 1)
        sel = (idx_ref[:, 0:1] == e_ids) | (idx_ref[:, 1:2] == e_ids)
        gated = jnp.where(sel, p, 0.0)
        wgt_ref[:, :] = gated / jnp.sum(gated, axis=-1, keepdims=True)
        xbf_ref[:, :] = x_ref[:, :].astype(jnp.bfloat16)
        out_ref[:, :] = jnp.zeros((n_tok, h), jnp.float32)

        def gated_x(origin, j):
            ids = lax.broadcasted_iota(jnp.int32, (n_tok, N_EXP), 1)
            e = origin * e_per + j
            g = jnp.sum(jnp.where(ids == e, wgt_ref[:, :], 0.0),
                        axis=1, keepdims=True)
            return xbf_ref[:, :] * g.astype(jnp.bfloat16)

        def contrib1(origin, slot, j):
            out_ref[:, :] = out_ref[:, :] + jnp.dot(
                gated_x(origin, j), comm_ref[slot, j, :, :],
                preferred_element_type=jnp.float32)

        def contrib_n(origin, slot, j0, nj):
            xg = jnp.concatenate(
                [gated_x(origin, j0 + jj) for jj in range(nj)], axis=1)
            w = comm_ref[slot, pl.ds(j0, nj), :, :].reshape(nj * d, h)
            out_ref[:, :] = out_ref[:, :] + jnp.dot(
                xg, w, preferred_element_type=jnp.float32)

        contrib_n(my, 0, 0, e_per)

        b_sends = []
        for j in (0, 1):
            recv_l[j].wait_recv()
            br = copy(1, j, 3, j, 8 + j, right)
            br.start()
            b_sends.append(br)
        for j in (3, 2):
            recv_r[j].wait_recv()
            bl = copy(2, j, 3, j, 8 + j, left)
            bl.start()
            b_sends.append(bl)

        opp = lax.rem(my + 2, N_DEV)
        contrib_n(left, 1, 0, 2)
        contrib_n(right, 2, 2, 2)
        recv_l[2].wait_recv()
        recv_l[3].wait_recv()
        contrib_n(left, 1, 2, 2)
        recv_r[1].wait_recv()
        recv_r[0].wait_recv()
        contrib_n(right, 2, 0, 2)
        recv_opp[0].wait_recv()
        contrib1(opp, 3, 0)
        recv_opp[3].wait_recv()
        contrib1(opp, 3, 3)
        recv_opp[1].wait_recv()
        contrib1(opp, 3, 1)
        recv_opp[2].wait_recv()
        contrib1(opp, 3, 2)

        for s in a_sends + b_sends:
            s.wait_send()

    return pl.pallas_call(
        body,
        out_shape=jax.ShapeDtypeStruct((n_tok, h), jnp.float32),
        in_specs=[
            pl.BlockSpec(memory_space=pltpu.VMEM),
            pl.BlockSpec(memory_space=pltpu.VMEM),
            pl.BlockSpec(memory_space=pltpu.VMEM),
            pl.BlockSpec(memory_space=pltpu.VMEM),
        ],
        out_specs=pl.BlockSpec(memory_space=pltpu.VMEM),
        scratch_shapes=[
            pltpu.VMEM((N_DEV, e_per, d, h), jnp.bfloat16),
            pltpu.VMEM((n_tok, N_EXP), jnp.float32),
            pltpu.VMEM((n_tok, d), jnp.bfloat16),
            pltpu.SemaphoreType.DMA((12,)),
            pltpu.SemaphoreType.DMA((12,)),
        ],
        compiler_params=pltpu.CompilerParams(collective_id=0),
    )(x, router_W, route_idx, expert_W.astype(jnp.bfloat16))


# device time: 84829 ns/iter; 1.9490x vs baseline; 1.0264x over previous
import jax
import jax.numpy as jnp
from jax import lax
from jax.experimental import pallas as pl
from jax.experimental.pallas import tpu as pltpu

N_DEV = 4
N_EXP = 16


def kernel(x, router_W, route_idx, expert_W):
    n_tok, d = x.shape
    e_per, _, h = expert_W.shape

    def body(x_ref, rw_ref, idx_ref, ew_ref, out_ref,
             comm_ref, wgt_ref, xbf_ref, send_sems, recv_sems):
        my = lax.axis_index("i")
        left = lax.rem(my + (N_DEV - 1), N_DEV)
        right = lax.rem(my + 1, N_DEV)

        comm_ref[0, :, :, :] = ew_ref[:, :, :]

        barrier_sem = pltpu.get_barrier_semaphore()
        for nbr in (left, right):
            pl.semaphore_signal(
                barrier_sem, inc=1,
                device_id=(nbr,), device_id_type=pl.DeviceIdType.MESH,
            )
        pl.semaphore_wait(barrier_sem, 2)

        def copy(src_slot, src_j, dst_slot, dst_j, sem, dev):
            return pltpu.make_async_remote_copy(
                src_ref=comm_ref.at[src_slot, src_j],
                dst_ref=comm_ref.at[dst_slot, dst_j],
                send_sem=send_sems.at[sem], recv_sem=recv_sems.at[sem],
                device_id=(dev,), device_id_type=pl.DeviceIdType.MESH,
            )

        a_sends = []
        for j in range(e_per):
            ar = copy(0, j, 1, j, j, right)
            ar.start()
            a_sends.append(ar)
        for j in reversed(range(e_per)):
            al = copy(0, j, 2, j, 4 + j, left)
            al.start()
            a_sends.append(al)

        recv_l = [copy(1, j, 1, j, j, left) for j in range(e_per)]
        recv_r = [copy(2, j, 2, j, 4 + j, right) for j in range(e_per)]

        dh = d // 2

        def copy_half(src_slot, j, k, dev):
            return pltpu.make_async_remote_copy(
                src_ref=comm_ref.at[src_slot, j, pl.ds(k * dh, dh)],
                dst_ref=comm_ref.at[3, j, pl.ds(k * dh, dh)],
                send_sem=send_sems.at[8 + 2 * j + k],
                recv_sem=recv_sems.at[8 + 2 * j + k],
                device_id=(dev,), device_id_type=pl.DeviceIdType.MESH,
            )

        recv_opp = [[copy_half(3, j, k, left) for k in range(2)]
                    for j in range(e_per)]

        scores = jnp.dot(x_ref[:, :], rw_ref[:, :],
                         preferred_element_type=jnp.float32)
        smax = jnp.max(scores, axis=-1, keepdims=True)
        p = jnp.exp(scores - smax)
        p = p / jnp.sum(p, axis=-1, keepdims=True)
        e_ids = lax.broadcasted_iota(jnp.int32, (n_tok, N_EXP), 1)
        sel = (idx_ref[:, 0:1] == e_ids) | (idx_ref[:, 1:2] == e_ids)
        gated = jnp.where(sel, p, 0.0)
        wgt_ref[:, :] = gated / jnp.sum(gated, axis=-1, keepdims=True)
        xbf_ref[:, :] = x_ref[:, :].astype(jnp.bfloat16)
        out_ref[:, :] = jnp.zeros((n_tok, h), jnp.float32)

        def gated_x(origin, j):
            ids = lax.broadcasted_iota(jnp.int32, (n_tok, N_EXP), 1)
            e = origin * e_per + j
            g = jnp.sum(jnp.where(ids == e, wgt_ref[:, :], 0.0),
                        axis=1, keepdims=True)
            return xbf_ref[:, :] * g.astype(jnp.bfloat16)

        def contrib_half(origin, slot, j, k):
            xg = gated_x(origin, j)[:, k * dh:(k + 1) * dh]
            out_ref[:, :] = out_ref[:, :] + jnp.dot(
                xg, comm_ref[slot, j, pl.ds(k * dh, dh), :],
                preferred_element_type=jnp.float32)

        def contrib_n(origin, slot, j0, nj):
            xg = jnp.concatenate(
                [gated_x(origin, j0 + jj) for jj in range(nj)], axis=1)
            w = comm_ref[slot, pl.ds(j0, nj), :, :].reshape(nj * d, h)
            out_ref[:, :] = out_ref[:, :] + jnp.dot(
                xg, w, preferred_element_type=jnp.float32)

        contrib_n(my, 0, 0, e_per)

        b_sends = []
        for j in (0, 1):
            recv_l[j].wait_recv()
            for k in range(2):
                br = copy_half(1, j, k, right)
                br.start()
                b_sends.append(br)
        for j in (3, 2):
            recv_r[j].wait_recv()
            for k in range(2):
                bl = copy_half(2, j, k, left)
                bl.start()
                b_sends.append(bl)

        opp = lax.rem(my + 2, N_DEV)
        contrib_n(left, 1, 0, 2)
        contrib_n(right, 2, 2, 2)
        recv_l[2].wait_recv()
        recv_l[3].wait_recv()
        contrib_n(left, 1, 2, 2)
        recv_r[1].wait_recv()
        recv_r[0].wait_recv()
        contrib_n(right, 2, 0, 2)
        for j_r, j_l in ((0, 3), (1, 2)):
            for k in range(2):
                recv_opp[j_r][k].wait_recv()
                contrib_half(opp, 3, j_r, k)
                recv_opp[j_l][k].wait_recv()
                contrib_half(opp, 3, j_l, k)

        for s in a_sends + b_sends:
            s.wait_send()

    return pl.pallas_call(
        body,
        out_shape=jax.ShapeDtypeStruct((n_tok, h), jnp.float32),
        in_specs=[
            pl.BlockSpec(memory_space=pltpu.VMEM),
            pl.BlockSpec(memory_space=pltpu.VMEM),
            pl.BlockSpec(memory_space=pltpu.VMEM),
            pl.BlockSpec(memory_space=pltpu.VMEM),
        ],
        out_specs=pl.BlockSpec(memory_space=pltpu.VMEM),
        scratch_shapes=[
            pltpu.VMEM((N_DEV, e_per, d, h), jnp.bfloat16),
            pltpu.VMEM((n_tok, N_EXP), jnp.float32),
            pltpu.VMEM((n_tok, d), jnp.bfloat16),
            pltpu.SemaphoreType.DMA((16,)),
            pltpu.SemaphoreType.DMA((16,)),
        ],
        compiler_params=pltpu.CompilerParams(collective_id=0),
    )(x, router_W, route_idx, expert_W.astype(jnp.bfloat16))
